# baseline (speedup 1.0000x reference)
import functools

import jax
import jax.numpy as jnp
from jax import lax
from jax.experimental import pallas as pl
from jax.experimental.pallas import tpu as pltpu
from jax.experimental.pallas import tpu_sc as plsc

N = 10000
E = 320000
F_IN = 128
H = 256
C = 64

NC = 2
NS = 16
L = 16
W128 = 128

CHUNK = 128
SECT = 16
EPT = E // NS
CPT = 160
NSECT = CPT // SECT
HSECT = NSECT // 2
EPT_PAD = CPT * CHUNK
DUMP = N
DPAD = 96
NROWS = N + DPAD
RPT = 624
TAIL = N - RPT * NS
BR = 1000
GRID = N // BR


def _fill(buf, d2, value):
    vals = jnp.full((L,), value, jnp.float32)

    def row(i, carry):
        for k in range(d2 // L):
            buf[i, pl.ds(k * L, L)] = vals
        return carry

    lax.fori_loop(0, CHUNK, row, 0)


def _zero_acc(zbuf, acc, s):
    zr = zbuf.shape[0]

    def span(base, nrows):
        for k in range(nrows // zr):
            pltpu.sync_copy(zbuf, acc.at[pl.ds(base + k * zr, zr)])
        rem = nrows % zr
        if rem:
            pltpu.sync_copy(zbuf.at[pl.ds(0, rem)],
                            acc.at[pl.ds(base + (nrows // zr) * zr, rem)])

    span(s * RPT, RPT)

    @pl.when(s == NS - 1)
    def _():
        span(RPT * NS, NROWS - RPT * NS)


def _copy_out(acc, out_hbm, c, s):
    pltpu.sync_copy(acc.at[pl.ds(s * RPT, RPT)],
                    out_hbm.at[c, pl.ds(s * RPT, RPT)])

    @pl.when(s == NS - 1)
    def _():
        pltpu.sync_copy(acc.at[pl.ds(RPT * NS, TAIL)],
                        out_hbm.at[c, pl.ds(RPT * NS, TAIL)])


def _make_agg(row_split):
    mesh = plsc.VectorSubcoreMesh(core_axis_name="c", subcore_axis_name="s")

    def body(src_hbm, dst_hbm, h_hbm, out_hbm,
             srcA, srcB, dstA, dstB, buf0, buf1, acc, semg0, semg1, sems):
        c = lax.axis_index("c")
        s = lax.axis_index("s")
        srcsec = (srcA, srcB)
        dstsec = (dstA, dstB)
        bufs = (buf0, buf1)
        semg = (semg0, semg1)
        nsect = HSECT if row_split else NSECT

        def sec_off(sec):
            if row_split:
                return pl.multiple_of((c * HSECT + sec) * SECT, SECT)
            return sec * SECT

        def src_slab(sec):
            return src_hbm.at[s, pl.ds(sec_off(sec), SECT)]

        def add_core_offset(sv):
            if row_split:
                return
            off = jnp.broadcast_to((c * N).astype(jnp.int32), (L,))

            def row(i, carry):
                for k in range(CHUNK // L):
                    sv[i, pl.ds(k * L, L)] = sv[i, pl.ds(k * L, L)] + off
                return carry

            lax.fori_loop(0, SECT, row, 0)

        def dst_slab(sec):
            return dst_hbm.at[s, pl.ds(sec_off(sec), SECT)]

        pltpu.sync_copy(src_slab(0), srcA)
        pltpu.sync_copy(dst_slab(0), dstA)
        add_core_offset(srcA)
        _fill(buf0, W128, 0.0)
        _zero_acc(buf0, acc, s)
        plsc.subcore_barrier()

        for sec in range(nsect):
            cur = sec % 2
            sv = srcsec[cur]
            dv = dstsec[cur]
            if sec + 1 < nsect:
                pltpu.async_copy(src_slab(sec + 1), srcsec[1 - cur], sems)
                pltpu.async_copy(dst_slab(sec + 1), dstsec[1 - cur], sems)

            def fire(j, b, sv=sv):
                pltpu.async_copy(h_hbm.at[sv.at[j]], bufs[b], semg[b])

            def wait(j, b, sv=sv):
                pltpu.make_async_copy(h_hbm.at[sv.at[j]], bufs[b],
                                      semg[b]).wait()

            fire(0, 0)
            fire(1, 1)

            def inner(jj, carry, fire=fire, wait=wait, dv=dv):
                for b in range(2):
                    j = jj * 2 + b
                    wait(j, b)
                    pltpu.sync_copy(bufs[b], acc.at[dv.at[j]], add=True)

                    @pl.when(j + 2 < SECT)
                    def _():
                        fire(j + 2, b)
                return carry

            lax.fori_loop(0, SECT // 2, inner, 0)
            if sec + 1 < nsect:
                pltpu.make_async_copy(src_slab(sec + 1), srcsec[1 - cur],
                                      sems).wait()
                pltpu.make_async_copy(dst_slab(sec + 1), dstsec[1 - cur],
                                      sems).wait()
                add_core_offset(srcsec[1 - cur])

        plsc.subcore_barrier()
        _copy_out(acc, out_hbm, c, s)

    return pl.kernel(
        body,
        out_type=jax.ShapeDtypeStruct((NC, N, W128), jnp.float32),
        mesh=mesh,
        scratch_types=[
            pltpu.VMEM((SECT, CHUNK), jnp.int32),
            pltpu.VMEM((SECT, CHUNK), jnp.int32),
            pltpu.VMEM((SECT, CHUNK), jnp.int32),
            pltpu.VMEM((SECT, CHUNK), jnp.int32),
            pltpu.VMEM((CHUNK, W128), jnp.float32),
            pltpu.VMEM((CHUNK, W128), jnp.float32),
            pltpu.VMEM_SHARED((NROWS, W128), jnp.float32),
            pltpu.SemaphoreType.DMA,
            pltpu.SemaphoreType.DMA,
            pltpu.SemaphoreType.DMA,
        ],
    )


def _make_deg():
    mesh = plsc.VectorSubcoreMesh(core_axis_name="c", subcore_axis_name="s")
    half = CPT // 2

    def body(dst_hbm, out_hbm, dst_v, ones_v, acc):
        c = lax.axis_index("c")
        s = lax.axis_index("s")
        pltpu.sync_copy(dst_hbm.at[s], dst_v)
        _fill(ones_v, L, 0.0)
        _zero_acc(ones_v, acc, s)
        _fill(ones_v, L, 1.0)
        plsc.subcore_barrier()

        def chunk(j, carry):
            pltpu.sync_copy(ones_v, acc.at[dst_v.at[j]], add=True)
            return carry

        lax.fori_loop(c * half, (c + 1) * half, chunk, 0)
        plsc.subcore_barrier()
        _copy_out(acc, out_hbm, c, s)

    return pl.kernel(
        body,
        out_type=jax.ShapeDtypeStruct((NC, N, L), jnp.float32),
        mesh=mesh,
        scratch_types=[
            pltpu.VMEM((CPT, CHUNK), jnp.int32),
            pltpu.VMEM((CHUNK, L), jnp.float32),
            pltpu.VMEM_SHARED((NROWS, L), jnp.float32),
        ],
    )


@functools.cache
def _get_agg(row_split):
    return _make_agg(row_split)


@functools.cache
def _get_deg():
    return _make_deg()


def _dinv_block(deg_ref):
    deg = deg_ref[0, :, 0:1] + deg_ref[1, :, 0:1] + 1.0
    return lax.rsqrt(deg)


def _k1_body(deg_ref, x_ref, w_ref, out_ref):
    dinv = _dinv_block(deg_ref)
    h = jnp.dot(x_ref[...], w_ref[...], preferred_element_type=jnp.float32)
    hs = h * dinv
    out_ref[0] = hs[:, :H // 2]
    out_ref[1] = hs[:, H // 2:]


def _k2_body(deg_ref, agg_ref, hs_ref, b_ref, w_ref, out_ref, x1_ref):
    dinv = _dinv_block(deg_ref)
    m = jnp.concatenate([agg_ref[0] + hs_ref[0], agg_ref[1] + hs_ref[1]],
                        axis=1)
    x1 = jax.nn.relu(dinv * m + b_ref[...])
    h2 = jnp.dot(x1, w_ref[...], preferred_element_type=jnp.float32)
    h2s = h2 * dinv
    out_ref[0] = h2s[:, :H // 2]
    out_ref[1] = h2s[:, H // 2:]
    x1_ref[...] = x1


def _k3_body(deg_ref, agg_ref, hs_ref, x1_ref, b_ref, w_ref, out_ref):
    dinv = _dinv_block(deg_ref)
    m = jnp.concatenate([agg_ref[0] + hs_ref[0], agg_ref[1] + hs_ref[1]],
                        axis=1)
    x2 = jax.nn.relu(dinv * m + b_ref[...]) + x1_ref[...]
    h3 = jnp.dot(x2, w_ref[...], preferred_element_type=jnp.float32)
    out_ref[...] = jnp.concatenate(
        [h3, jnp.zeros((BR, W128 - C), jnp.float32)], axis=1)


def _k4_body(agg_ref, h3_ref, b_ref, out_ref):
    out_ref[...] = (agg_ref[0, :, :C] + agg_ref[1, :, :C]
                    + h3_ref[:, :C] + b_ref[...])


def _spec_deg():
    return pl.BlockSpec((NC, BR, L), lambda i: (0, i, 0))


def _spec_half(d2):
    return pl.BlockSpec((NC, BR, d2), lambda i: (0, i, 0))


def _spec_full(shape):
    return pl.BlockSpec(shape, lambda i: tuple(0 for _ in shape))


def _tc_call(body, in_specs, out_specs, out_shape):
    return pl.pallas_call(
        body,
        grid=(GRID,),
        in_specs=in_specs,
        out_specs=out_specs,
        out_shape=out_shape,
    )


def kernel(x, edge_index, W1, b1, W2, b2, W3, b3):
    src = edge_index[0]
    dst = edge_index[1]
    pads = (jnp.arange(EPT_PAD - EPT, dtype=jnp.int32) * 521) % N
    srcr = jnp.concatenate(
        [src.reshape(NS, EPT), jnp.broadcast_to(pads, (NS, EPT_PAD - EPT))],
        axis=1)
    src3 = srcr.reshape(NS, CPT, CHUNK)
    padv = DUMP + (jnp.arange(EPT_PAD - EPT, dtype=jnp.int32) % DPAD)
    dst3 = jnp.concatenate(
        [dst.reshape(NS, EPT),
         jnp.broadcast_to(padv, (NS, EPT_PAD - EPT))],
        axis=1).reshape(NS, CPT, CHUNK)

    b1r = b1.reshape(1, H)
    b2r = b2.reshape(1, H)
    b3r = b3.reshape(1, C)

    deg = _get_deg()(dst3)

    h1s = _tc_call(
        _k1_body,
        [_spec_deg(),
         pl.BlockSpec((BR, F_IN), lambda i: (i, 0)),
         _spec_full((F_IN, H))],
        _spec_half(H // 2),
        jax.ShapeDtypeStruct((NC, N, H // 2), jnp.float32),
    )(deg, x, W1)

    agg1 = _get_agg(False)(src3, dst3, h1s.reshape(NC * N, H // 2))

    h2s, x1 = _tc_call(
        _k2_body,
        [_spec_deg(), _spec_half(H // 2), _spec_half(H // 2),
         _spec_full((1, H)), _spec_full((H, H))],
        [_spec_half(H // 2), pl.BlockSpec((BR, H), lambda i: (i, 0))],
        [jax.ShapeDtypeStruct((NC, N, H // 2), jnp.float32),
         jax.ShapeDtypeStruct((N, H), jnp.float32)],
    )(deg, agg1, h1s, b1r, W2)

    agg2 = _get_agg(False)(src3, dst3, h2s.reshape(NC * N, H // 2))

    h3 = _tc_call(
        _k3_body,
        [_spec_deg(), _spec_half(H // 2), _spec_half(H // 2),
         pl.BlockSpec((BR, H), lambda i: (i, 0)),
         _spec_full((1, H)), _spec_full((H, C))],
        pl.BlockSpec((BR, W128), lambda i: (i, 0)),
        jax.ShapeDtypeStruct((N, W128), jnp.float32),
    )(deg, agg2, h2s, x1, b2r, W3)

    agg3 = _get_agg(True)(src3, dst3, h3)

    out = _tc_call(
        _k4_body,
        [_spec_half(W128), pl.BlockSpec((BR, W128), lambda i: (i, 0)),
         _spec_full((1, C))],
        pl.BlockSpec((BR, C), lambda i: (i, 0)),
        jax.ShapeDtypeStruct((N, C), jnp.float32),
    )(agg3, h3, b3r)

    return out


# speedup vs baseline: 1.0124x; 1.0124x over previous
import functools

import jax
import jax.numpy as jnp
from jax import lax
from jax.experimental import pallas as pl
from jax.experimental.pallas import tpu as pltpu
from jax.experimental.pallas import tpu_sc as plsc

N = 10000
E = 320000
F_IN = 128
H = 256
C = 64

NC = 2
NS = 16
L = 16
W128 = 128

CHUNK = 128
SECT = 16
EPT = E // NS
CPT = 160
NSECT = CPT // SECT
HSECT = NSECT // 2
EPT_PAD = CPT * CHUNK
DUMP = N
DPAD = 96
NROWS = N + DPAD
RPT = 624
TAIL = N - RPT * NS
BR = 2000
GRID = N // BR


def _fill(buf, d2, value):
    vals = jnp.full((L,), value, jnp.float32)

    def row(i, carry):
        for k in range(d2 // L):
            buf[i, pl.ds(k * L, L)] = vals
        return carry

    lax.fori_loop(0, CHUNK, row, 0)


def _zero_acc(zbuf, acc, s):
    zr = zbuf.shape[0]

    def span(base, nrows):
        for k in range(nrows // zr):
            pltpu.sync_copy(zbuf, acc.at[pl.ds(base + k * zr, zr)])
        rem = nrows % zr
        if rem:
            pltpu.sync_copy(zbuf.at[pl.ds(0, rem)],
                            acc.at[pl.ds(base + (nrows // zr) * zr, rem)])

    span(s * RPT, RPT)

    @pl.when(s == NS - 1)
    def _():
        span(RPT * NS, NROWS - RPT * NS)


def _copy_out(acc, out_hbm, c, s):
    pltpu.sync_copy(acc.at[pl.ds(s * RPT, RPT)],
                    out_hbm.at[c, pl.ds(s * RPT, RPT)])

    @pl.when(s == NS - 1)
    def _():
        pltpu.sync_copy(acc.at[pl.ds(RPT * NS, TAIL)],
                        out_hbm.at[c, pl.ds(RPT * NS, TAIL)])


def _make_agg(row_split):
    mesh = plsc.VectorSubcoreMesh(core_axis_name="c", subcore_axis_name="s")

    def body(src_hbm, dst_hbm, h_hbm, out_hbm,
             srcA, srcB, dstA, dstB, buf0, buf1, acc, semg0, semg1, sems):
        c = lax.axis_index("c")
        s = lax.axis_index("s")
        srcsec = (srcA, srcB)
        dstsec = (dstA, dstB)
        bufs = (buf0, buf1)
        semg = (semg0, semg1)
        nsect = HSECT if row_split else NSECT

        def sec_off(sec):
            if row_split:
                return pl.multiple_of((c * HSECT + sec) * SECT, SECT)
            return sec * SECT

        def src_slab(sec):
            return src_hbm.at[s, pl.ds(sec_off(sec), SECT)]

        def add_core_offset(sv):
            if row_split:
                return
            off = jnp.broadcast_to((c * N).astype(jnp.int32), (L,))

            def row(i, carry):
                for k in range(CHUNK // L):
                    sv[i, pl.ds(k * L, L)] = sv[i, pl.ds(k * L, L)] + off
                return carry

            lax.fori_loop(0, SECT, row, 0)

        def dst_slab(sec):
            return dst_hbm.at[s, pl.ds(sec_off(sec), SECT)]

        pltpu.sync_copy(src_slab(0), srcA)
        pltpu.sync_copy(dst_slab(0), dstA)
        add_core_offset(srcA)
        _fill(buf0, W128, 0.0)
        _zero_acc(buf0, acc, s)
        plsc.subcore_barrier()

        for sec in range(nsect):
            cur = sec % 2
            sv = srcsec[cur]
            dv = dstsec[cur]
            if sec + 1 < nsect:
                pltpu.async_copy(src_slab(sec + 1), srcsec[1 - cur], sems)
                pltpu.async_copy(dst_slab(sec + 1), dstsec[1 - cur], sems)

            def fire(j, b, sv=sv):
                pltpu.async_copy(h_hbm.at[sv.at[j]], bufs[b], semg[b])

            def wait(j, b, sv=sv):
                pltpu.make_async_copy(h_hbm.at[sv.at[j]], bufs[b],
                                      semg[b]).wait()

            fire(0, 0)
            fire(1, 1)

            def inner(jj, carry, fire=fire, wait=wait, dv=dv):
                for b in range(2):
                    j = jj * 2 + b
                    wait(j, b)
                    pltpu.sync_copy(bufs[b], acc.at[dv.at[j]], add=True)

                    @pl.when(j + 2 < SECT)
                    def _():
                        fire(j + 2, b)
                return carry

            lax.fori_loop(0, SECT // 2, inner, 0)
            if sec + 1 < nsect:
                pltpu.make_async_copy(src_slab(sec + 1), srcsec[1 - cur],
                                      sems).wait()
                pltpu.make_async_copy(dst_slab(sec + 1), dstsec[1 - cur],
                                      sems).wait()
                add_core_offset(srcsec[1 - cur])

        plsc.subcore_barrier()
        _copy_out(acc, out_hbm, c, s)

    return pl.kernel(
        body,
        out_type=jax.ShapeDtypeStruct((NC, N, W128), jnp.float32),
        mesh=mesh,
        scratch_types=[
            pltpu.VMEM((SECT, CHUNK), jnp.int32),
            pltpu.VMEM((SECT, CHUNK), jnp.int32),
            pltpu.VMEM((SECT, CHUNK), jnp.int32),
            pltpu.VMEM((SECT, CHUNK), jnp.int32),
            pltpu.VMEM((CHUNK, W128), jnp.float32),
            pltpu.VMEM((CHUNK, W128), jnp.float32),
            pltpu.VMEM_SHARED((NROWS, W128), jnp.float32),
            pltpu.SemaphoreType.DMA,
            pltpu.SemaphoreType.DMA,
            pltpu.SemaphoreType.DMA,
        ],
    )


def _make_deg():
    mesh = plsc.VectorSubcoreMesh(core_axis_name="c", subcore_axis_name="s")
    half = CPT // 2

    def body(dst_hbm, out_hbm, dst_v, ones_v, acc):
        c = lax.axis_index("c")
        s = lax.axis_index("s")
        pltpu.sync_copy(dst_hbm.at[s], dst_v)
        _fill(ones_v, L, 0.0)
        _zero_acc(ones_v, acc, s)
        _fill(ones_v, L, 1.0)
        plsc.subcore_barrier()

        def chunk(j, carry):
            pltpu.sync_copy(ones_v, acc.at[dst_v.at[j]], add=True)
            return carry

        lax.fori_loop(c * half, (c + 1) * half, chunk, 0)
        plsc.subcore_barrier()
        _copy_out(acc, out_hbm, c, s)

    return pl.kernel(
        body,
        out_type=jax.ShapeDtypeStruct((NC, N, L), jnp.float32),
        mesh=mesh,
        scratch_types=[
            pltpu.VMEM((CPT, CHUNK), jnp.int32),
            pltpu.VMEM((CHUNK, L), jnp.float32),
            pltpu.VMEM_SHARED((NROWS, L), jnp.float32),
        ],
    )


@functools.cache
def _get_agg(row_split):
    return _make_agg(row_split)


@functools.cache
def _get_deg():
    return _make_deg()


def _dinv_block(deg_ref):
    deg = deg_ref[0, :, 0:1] + deg_ref[1, :, 0:1] + 1.0
    return lax.rsqrt(deg)


def _k1_body(deg_ref, x_ref, w_ref, out_ref):
    dinv = _dinv_block(deg_ref)
    h = jnp.dot(x_ref[...], w_ref[...], preferred_element_type=jnp.float32)
    hs = h * dinv
    out_ref[0] = hs[:, :H // 2]
    out_ref[1] = hs[:, H // 2:]


def _k2_body(deg_ref, agg_ref, hs_ref, b_ref, w_ref, out_ref, x1_ref):
    dinv = _dinv_block(deg_ref)
    m = jnp.concatenate([agg_ref[0] + hs_ref[0], agg_ref[1] + hs_ref[1]],
                        axis=1)
    x1 = jax.nn.relu(dinv * m + b_ref[...])
    h2 = jnp.dot(x1, w_ref[...], preferred_element_type=jnp.float32)
    h2s = h2 * dinv
    out_ref[0] = h2s[:, :H // 2]
    out_ref[1] = h2s[:, H // 2:]
    x1_ref[...] = x1


def _k3_body(deg_ref, agg_ref, hs_ref, x1_ref, b_ref, w_ref, out_ref):
    dinv = _dinv_block(deg_ref)
    m = jnp.concatenate([agg_ref[0] + hs_ref[0], agg_ref[1] + hs_ref[1]],
                        axis=1)
    x2 = jax.nn.relu(dinv * m + b_ref[...]) + x1_ref[...]
    h3 = jnp.dot(x2, w_ref[...], preferred_element_type=jnp.float32)
    out_ref[...] = jnp.concatenate(
        [h3, jnp.zeros((BR, W128 - C), jnp.float32)], axis=1)


def _k4_body(agg_ref, h3_ref, b_ref, out_ref):
    out_ref[...] = (agg_ref[0, :, :C] + agg_ref[1, :, :C]
                    + h3_ref[:, :C] + b_ref[...])


def _spec_deg():
    return pl.BlockSpec((NC, BR, L), lambda i: (0, i, 0))


def _spec_half(d2):
    return pl.BlockSpec((NC, BR, d2), lambda i: (0, i, 0))


def _spec_full(shape):
    return pl.BlockSpec(shape, lambda i: tuple(0 for _ in shape))


def _tc_call(body, in_specs, out_specs, out_shape):
    return pl.pallas_call(
        body,
        grid=(GRID,),
        in_specs=in_specs,
        out_specs=out_specs,
        out_shape=out_shape,
    )


def kernel(x, edge_index, W1, b1, W2, b2, W3, b3):
    src = edge_index[0]
    dst = edge_index[1]
    pads = (jnp.arange(EPT_PAD - EPT, dtype=jnp.int32) * 521) % N
    srcr = jnp.concatenate(
        [src.reshape(NS, EPT), jnp.broadcast_to(pads, (NS, EPT_PAD - EPT))],
        axis=1)
    src3 = srcr.reshape(NS, CPT, CHUNK)
    padv = DUMP + (jnp.arange(EPT_PAD - EPT, dtype=jnp.int32) % DPAD)
    dst3 = jnp.concatenate(
        [dst.reshape(NS, EPT),
         jnp.broadcast_to(padv, (NS, EPT_PAD - EPT))],
        axis=1).reshape(NS, CPT, CHUNK)

    b1r = b1.reshape(1, H)
    b2r = b2.reshape(1, H)
    b3r = b3.reshape(1, C)

    deg = _get_deg()(dst3)

    h1s = _tc_call(
        _k1_body,
        [_spec_deg(),
         pl.BlockSpec((BR, F_IN), lambda i: (i, 0)),
         _spec_full((F_IN, H))],
        _spec_half(H // 2),
        jax.ShapeDtypeStruct((NC, N, H // 2), jnp.float32),
    )(deg, x, W1)

    agg1 = _get_agg(False)(src3, dst3, h1s.reshape(NC * N, H // 2))

    h2s, x1 = _tc_call(
        _k2_body,
        [_spec_deg(), _spec_half(H // 2), _spec_half(H // 2),
         _spec_full((1, H)), _spec_full((H, H))],
        [_spec_half(H // 2), pl.BlockSpec((BR, H), lambda i: (i, 0))],
        [jax.ShapeDtypeStruct((NC, N, H // 2), jnp.float32),
         jax.ShapeDtypeStruct((N, H), jnp.float32)],
    )(deg, agg1, h1s, b1r, W2)

    agg2 = _get_agg(False)(src3, dst3, h2s.reshape(NC * N, H // 2))

    h3 = _tc_call(
        _k3_body,
        [_spec_deg(), _spec_half(H // 2), _spec_half(H // 2),
         pl.BlockSpec((BR, H), lambda i: (i, 0)),
         _spec_full((1, H)), _spec_full((H, C))],
        pl.BlockSpec((BR, W128), lambda i: (i, 0)),
        jax.ShapeDtypeStruct((N, W128), jnp.float32),
    )(deg, agg2, h2s, x1, b2r, W3)

    agg3 = _get_agg(True)(src3, dst3, h3)

    out = _tc_call(
        _k4_body,
        [_spec_half(W128), pl.BlockSpec((BR, W128), lambda i: (i, 0)),
         _spec_full((1, C))],
        pl.BlockSpec((BR, C), lambda i: (i, 0)),
        jax.ShapeDtypeStruct((N, C), jnp.float32),
    )(agg3, h3, b3r)

    return out
